# Initial kernel scaffold; baseline (speedup 1.0000x reference)
import functools

import jax
import jax.numpy as jnp
from jax import lax
from jax.experimental import pallas as pl
from jax.experimental.pallas import tpu as pltpu
from jax.experimental.pallas import tpu_sc as plsc

N = 10000
IN = 128
HID = 32
HEADS = 4
OUT = 64
E = 320000

NC = 2
NS = 16
NW = NC * NS
LANES = 16

N_PAD = 10016
ROWS_PER_TILE = N_PAD // NS

B = 128
NE_REAL = E + N
CHUNK = -(-NE_REAL // (NW * B)) * B
E_PAD = CHUNK * NW
NBLOCKS = CHUNK // B


def _sc_edge_kernel(HC, H, R):
    mesh = plsc.VectorSubcoreMesh(core_axis_name="c", subcore_axis_name="s")
    C = HC // H
    nfull = ROWS_PER_TILE // B
    nrem = ROWS_PER_TILE % B

    @functools.partial(
        pl.kernel,
        mesh=mesh,
        out_type=jax.ShapeDtypeStruct((NC, N_PAD, R), jnp.float32),
        scratch_types=[
            pltpu.VMEM((B,), jnp.int32),
            pltpu.VMEM((B,), jnp.int32),
            pltpu.VMEM((B, 16), jnp.float32),
            pltpu.VMEM((B, 16), jnp.float32),
            pltpu.VMEM((B, HC), jnp.float32),
            pltpu.VMEM((B, R), jnp.float32),
            pltpu.VMEM_SHARED((N_PAD, R), jnp.float32),
            pltpu.SemaphoreType.DMA,
            pltpu.SemaphoreType.DMA,
            pltpu.SemaphoreType.DMA,
        ],
    )
    def ker(src_hbm, dst_hbm, tsrc_hbm, tdst_hbm, h_hbm, out_hbm,
            srcb, dstb, tsb, tdb, hb, msgb, acc, sem0, sem1, sem2):
        cid = lax.axis_index("c")
        sid = lax.axis_index("s")
        wid = sid * NC + cid

        zero16 = jnp.zeros((LANES,), jnp.float32)
        headmask = lax.iota(jnp.int32, LANES) < H

        @pl.loop(0, B)
        def _(e):
            @pl.loop(0, R, step=LANES)
            def _(cc):
                msgb[e, pl.ds(cc, LANES)] = zero16

        row0 = sid * ROWS_PER_TILE

        @pl.loop(0, nfull)
        def _(k):
            pltpu.sync_copy(msgb, acc.at[pl.ds(row0 + k * B, B)])

        if nrem:
            pltpu.sync_copy(msgb.at[pl.ds(0, nrem)],
                            acc.at[pl.ds(row0 + nfull * B, nrem)])

        plsc.subcore_barrier()

        base0 = wid * CHUNK

        @pl.loop(0, NBLOCKS)
        def _(i):
            base = base0 + i * B
            pltpu.sync_copy(src_hbm.at[pl.ds(base, B)], srcb)
            pltpu.sync_copy(dst_hbm.at[pl.ds(base, B)], dstb)
            c0 = pltpu.async_copy(tsrc_hbm.at[srcb], tsb, sem0)
            c1 = pltpu.async_copy(tdst_hbm.at[dstb], tdb, sem1)
            c2 = pltpu.async_copy(h_hbm.at[srcb], hb, sem2)
            c0.wait()
            c1.wait()
            c2.wait()

            @pl.loop(0, B)
            def _(e):
                ev = tsb[e, pl.ds(0, LANES)] + tdb[e, pl.ds(0, LANES)]
                ev = jnp.maximum(ev, 0.2 * ev)
                wv = jnp.exp(ev)
                msgb[e, pl.ds(HC, LANES)] = jnp.where(headmask, wv, 0.0)
                for h in range(H):
                    wh = msgb[e, HC + h]
                    for q in range(C // LANES):
                        coff = h * C + q * LANES
                        msgb[e, pl.ds(coff, LANES)] = (
                            hb[e, pl.ds(coff, LANES)] * wh)

            pltpu.sync_copy(msgb, acc.at[dstb], add=True)

        plsc.subcore_barrier()

        @pl.loop(0, nfull)
        def _(k):
            pltpu.sync_copy(acc.at[pl.ds(row0 + k * B, B)],
                            out_hbm.at[cid, pl.ds(row0 + k * B, B)])

        if nrem:
            pltpu.sync_copy(acc.at[pl.ds(row0 + nfull * B, nrem)],
                            out_hbm.at[cid, pl.ds(row0 + nfull * B, nrem)])

    return ker


def _dense_kernel(x_pad, W, Asrc16, Adst16):
    HCo = W.shape[1]

    def body(x_ref, w_ref, as_ref, ad_ref, h_ref, ts_ref, td_ref):
        h = jnp.dot(x_ref[...], w_ref[...],
                    preferred_element_type=jnp.float32,
                    precision=lax.Precision.HIGHEST)
        h_ref[...] = h
        ts_ref[...] = jnp.dot(h, as_ref[...],
                              preferred_element_type=jnp.float32,
                              precision=lax.Precision.HIGHEST)
        td_ref[...] = jnp.dot(h, ad_ref[...],
                              preferred_element_type=jnp.float32,
                              precision=lax.Precision.HIGHEST)

    return pl.pallas_call(
        body,
        out_shape=(
            jax.ShapeDtypeStruct((N_PAD, HCo), jnp.float32),
            jax.ShapeDtypeStruct((N_PAD, 16), jnp.float32),
            jax.ShapeDtypeStruct((N_PAD, 16), jnp.float32),
        ),
    )(x_pad, W, Asrc16, Adst16)


def _combine_kernel(acc, P, Q, b2d, relu):
    HCo = P.shape[1]

    def body(a_ref, p_ref, q_ref, b_ref, o_ref):
        a = a_ref[0] + a_ref[1]
        num = jnp.dot(a, p_ref[...], preferred_element_type=jnp.float32,
                      precision=lax.Precision.HIGHEST)
        den = jnp.dot(a, q_ref[...], preferred_element_type=jnp.float32,
                      precision=lax.Precision.HIGHEST) + 1e-16
        o = num / den + b_ref[...]
        if relu:
            o = jnp.maximum(o, 0.0)
        o_ref[...] = o

    return pl.pallas_call(
        body,
        out_shape=jax.ShapeDtypeStruct((N_PAD, HCo), jnp.float32),
    )(acc, P, Q, b2d)


def _expand_attn(a, HC):
    H, C = a.shape
    mask = (jnp.arange(HC)[:, None] // C) == jnp.arange(H)[None, :]
    cols = jnp.where(mask, a.reshape(-1)[:, None], 0.0)
    return jnp.pad(cols, ((0, 0), (0, 16 - H)))


def _pq(HC, H, R):
    C = HC // H
    P = jnp.pad(jnp.eye(HC, dtype=jnp.float32), ((0, R - HC), (0, 0)))
    qrows = (jnp.arange(HC)[None, :] // C) == jnp.arange(H)[:, None]
    Q = jnp.zeros((R, HC), jnp.float32).at[HC:HC + H].set(qrows.astype(jnp.float32))
    return P, Q


_EDGE_L12 = _sc_edge_kernel(HEADS * HID, HEADS, 144)
_EDGE_L3 = _sc_edge_kernel(OUT, 1, 80)


def kernel(x, edge_index, W1, a1_src, a1_dst, b1, W2, a2_src, a2_dst, b2,
           W3, a3_src, a3_dst, b3):
    loop = jnp.arange(N, dtype=jnp.int32)
    src = jnp.concatenate([edge_index[0].astype(jnp.int32), loop])
    dst = jnp.concatenate([edge_index[1].astype(jnp.int32), loop])
    pad = jnp.full((E_PAD - NE_REAL,), N, jnp.int32)
    src = jnp.concatenate([src, pad])
    dst = jnp.concatenate([dst, pad])

    x_pad = jnp.pad(x, ((0, N_PAD - N), (0, 0)))

    h1, t1s, t1d = _dense_kernel(x_pad, W1, _expand_attn(a1_src, HEADS * HID),
                                 _expand_attn(a1_dst, HEADS * HID))
    acc1 = _EDGE_L12(src, dst, t1s, t1d, h1)
    P, Q = _pq(HEADS * HID, HEADS, 144)
    x2 = _combine_kernel(acc1, P, Q, b1.reshape(1, -1), relu=True)

    h2, t2s, t2d = _dense_kernel(x2, W2, _expand_attn(a2_src, HEADS * HID),
                                 _expand_attn(a2_dst, HEADS * HID))
    acc2 = _EDGE_L12(src, dst, t2s, t2d, h2)
    x3 = _combine_kernel(acc2, P, Q, b2.reshape(1, -1), relu=True)

    h3, t3s, t3d = _dense_kernel(x3, W3, _expand_attn(a3_src, OUT),
                                 _expand_attn(a3_dst, OUT))
    acc3 = _EDGE_L3(src, dst, t3s, t3d, h3)
    P3, Q3 = _pq(OUT, 1, 80)
    out = _combine_kernel(acc3, P3, Q3, b3.reshape(1, -1), relu=False)

    return out[:N]


# speedup vs baseline: 26.0916x; 26.0916x over previous
import functools

import jax
import jax.numpy as jnp
from jax import lax
from jax.experimental import pallas as pl
from jax.experimental.pallas import tpu as pltpu
from jax.experimental.pallas import tpu_sc as plsc

N = 10000
IN = 128
HID = 32
HEADS = 4
OUT = 64
E = 320000

NC = 2
NS = 16
NW = NC * NS
LANES = 16

N_PAD = 10112
ROWS_PER_TILE = N_PAD // NS

B = 128
NE_REAL = E + N
CHUNK = -(-NE_REAL // (NW * B)) * B
E_PAD = CHUNK * NW
NBLOCKS = CHUNK // B


def _sc_edge_kernel(HC, H, R):
    mesh = plsc.VectorSubcoreMesh(core_axis_name="c", subcore_axis_name="s")
    C = HC // H
    nfull = ROWS_PER_TILE // B
    nrem = ROWS_PER_TILE % B

    @functools.partial(
        pl.kernel,
        mesh=mesh,
        compiler_params=pltpu.CompilerParams(use_tc_tiling_on_sc=False),
        out_type=jax.ShapeDtypeStruct((NC, N_PAD, R), jnp.float32),
        scratch_types=[
            pltpu.VMEM((B,), jnp.int32),
            pltpu.VMEM((B,), jnp.int32),
            pltpu.VMEM((B, 16), jnp.float32),
            pltpu.VMEM((B, 16), jnp.float32),
            pltpu.VMEM((B, HC), jnp.float32),
            pltpu.VMEM((B, R), jnp.float32),
            pltpu.VMEM_SHARED((N_PAD, R), jnp.float32),
            pltpu.SemaphoreType.DMA,
            pltpu.SemaphoreType.DMA,
            pltpu.SemaphoreType.DMA,
        ],
    )
    def ker(src_hbm, dst_hbm, tsrc_hbm, tdst_hbm, h_hbm, out_hbm,
            srcb, dstb, tsb, tdb, hb, msgb, acc, sem0, sem1, sem2):
        cid = lax.axis_index("c")
        sid = lax.axis_index("s")
        wid = sid * NC + cid

        zero16 = jnp.zeros((LANES,), jnp.float32)
        headmask = lax.iota(jnp.int32, LANES) < H

        @pl.loop(0, B)
        def _(e):
            @pl.loop(0, R, step=LANES)
            def _(cc):
                msgb[e, pl.ds(cc, LANES)] = zero16

        row0 = sid * ROWS_PER_TILE

        @pl.loop(0, nfull)
        def _(k):
            pltpu.sync_copy(msgb, acc.at[pl.ds(row0 + k * B, B)])

        if nrem:
            pltpu.sync_copy(msgb.at[pl.ds(0, nrem)],
                            acc.at[pl.ds(row0 + nfull * B, nrem)])

        plsc.subcore_barrier()

        base0 = wid * CHUNK

        @pl.loop(0, NBLOCKS)
        def _(i):
            base = base0 + i * B
            pltpu.sync_copy(src_hbm.at[pl.ds(base, B)], srcb)
            pltpu.sync_copy(dst_hbm.at[pl.ds(base, B)], dstb)
            c0 = pltpu.async_copy(tsrc_hbm.at[srcb], tsb, sem0)
            c1 = pltpu.async_copy(tdst_hbm.at[dstb], tdb, sem1)
            c2 = pltpu.async_copy(h_hbm.at[srcb], hb, sem2)
            c0.wait()
            c1.wait()
            c2.wait()

            @pl.loop(0, B)
            def _(e):
                ev = tsb[e, pl.ds(0, LANES)] + tdb[e, pl.ds(0, LANES)]
                ev = jnp.maximum(ev, 0.2 * ev)
                wv = jnp.exp(ev)
                msgb[e, pl.ds(HC, LANES)] = jnp.where(headmask, wv, 0.0)
                for h in range(H):
                    wh = wv[h]
                    for q in range(C // LANES):
                        coff = h * C + q * LANES
                        msgb[e, pl.ds(coff, LANES)] = (
                            hb[e, pl.ds(coff, LANES)] * wh)

            pltpu.sync_copy(msgb, acc.at[dstb], add=True)

        plsc.subcore_barrier()

        @pl.loop(0, nfull)
        def _(k):
            pltpu.sync_copy(acc.at[pl.ds(row0 + k * B, B)],
                            out_hbm.at[cid, pl.ds(row0 + k * B, B)])

        if nrem:
            pltpu.sync_copy(acc.at[pl.ds(row0 + nfull * B, nrem)],
                            out_hbm.at[cid, pl.ds(row0 + nfull * B, nrem)])

    return ker


def _dense_kernel(x_pad, W, Asrc16, Adst16):
    HCo = W.shape[1]

    def body(x_ref, w_ref, as_ref, ad_ref, h_ref, ts_ref, td_ref):
        h = jnp.dot(x_ref[...], w_ref[...],
                    preferred_element_type=jnp.float32,
                    precision=lax.Precision.HIGHEST)
        h_ref[...] = h
        ts_ref[...] = jnp.dot(h, as_ref[...],
                              preferred_element_type=jnp.float32,
                              precision=lax.Precision.HIGHEST)
        td_ref[...] = jnp.dot(h, ad_ref[...],
                              preferred_element_type=jnp.float32,
                              precision=lax.Precision.HIGHEST)

    Bn = N_PAD // 8
    IK = x_pad.shape[1]
    return pl.pallas_call(
        body,
        grid=(8,),
        in_specs=[
            pl.BlockSpec((Bn, IK), lambda i: (i, 0)),
            pl.BlockSpec((IK, HCo), lambda i: (0, 0)),
            pl.BlockSpec((HCo, 16), lambda i: (0, 0)),
            pl.BlockSpec((HCo, 16), lambda i: (0, 0)),
        ],
        out_specs=(
            pl.BlockSpec((Bn, HCo), lambda i: (i, 0)),
            pl.BlockSpec((Bn, 16), lambda i: (i, 0)),
            pl.BlockSpec((Bn, 16), lambda i: (i, 0)),
        ),
        out_shape=(
            jax.ShapeDtypeStruct((N_PAD, HCo), jnp.float32),
            jax.ShapeDtypeStruct((N_PAD, 16), jnp.float32),
            jax.ShapeDtypeStruct((N_PAD, 16), jnp.float32),
        ),
    )(x_pad, W, Asrc16, Adst16)


def _combine_kernel(acc, P, Q, b2d, relu):
    HCo = P.shape[1]

    def body(a_ref, p_ref, q_ref, b_ref, o_ref):
        a = a_ref[0] + a_ref[1]
        num = jnp.dot(a, p_ref[...], preferred_element_type=jnp.float32,
                      precision=lax.Precision.HIGHEST)
        den = jnp.dot(a, q_ref[...], preferred_element_type=jnp.float32,
                      precision=lax.Precision.HIGHEST) + 1e-16
        o = num / den + b_ref[...]
        if relu:
            o = jnp.maximum(o, 0.0)
        o_ref[...] = o

    R = P.shape[0]
    Bn = N_PAD // 8
    return pl.pallas_call(
        body,
        grid=(8,),
        in_specs=[
            pl.BlockSpec((2, Bn, R), lambda i: (0, i, 0)),
            pl.BlockSpec((R, HCo), lambda i: (0, 0)),
            pl.BlockSpec((R, HCo), lambda i: (0, 0)),
            pl.BlockSpec((1, HCo), lambda i: (0, 0)),
        ],
        out_specs=pl.BlockSpec((Bn, HCo), lambda i: (i, 0)),
        out_shape=jax.ShapeDtypeStruct((N_PAD, HCo), jnp.float32),
    )(acc, P, Q, b2d)


def _expand_attn(a, HC):
    H, C = a.shape
    mask = (jnp.arange(HC)[:, None] // C) == jnp.arange(H)[None, :]
    cols = jnp.where(mask, a.reshape(-1)[:, None], 0.0)
    return jnp.pad(cols, ((0, 0), (0, 16 - H)))


def _pq(HC, H, R):
    C = HC // H
    P = jnp.pad(jnp.eye(HC, dtype=jnp.float32), ((0, R - HC), (0, 0)))
    qrows = (jnp.arange(HC)[None, :] // C) == jnp.arange(H)[:, None]
    Q = jnp.zeros((R, HC), jnp.float32).at[HC:HC + H].set(qrows.astype(jnp.float32))
    return P, Q


_EDGE_L12 = _sc_edge_kernel(HEADS * HID, HEADS, 144)
_EDGE_L3 = _sc_edge_kernel(OUT, 1, 80)


def kernel(x, edge_index, W1, a1_src, a1_dst, b1, W2, a2_src, a2_dst, b2,
           W3, a3_src, a3_dst, b3):
    loop = jnp.arange(N, dtype=jnp.int32)
    src = jnp.concatenate([edge_index[0].astype(jnp.int32), loop])
    dst = jnp.concatenate([edge_index[1].astype(jnp.int32), loop])
    pad = jnp.full((E_PAD - NE_REAL,), N, jnp.int32)
    src = jnp.concatenate([src, pad])
    dst = jnp.concatenate([dst, pad])

    x_pad = jnp.pad(x, ((0, N_PAD - N), (0, 0)))

    h1, t1s, t1d = _dense_kernel(x_pad, W1, _expand_attn(a1_src, HEADS * HID),
                                 _expand_attn(a1_dst, HEADS * HID))
    acc1 = _EDGE_L12(src, dst, t1s, t1d, h1)
    P, Q = _pq(HEADS * HID, HEADS, 144)
    x2 = _combine_kernel(acc1, P, Q, b1.reshape(1, -1), relu=True)

    h2, t2s, t2d = _dense_kernel(x2, W2, _expand_attn(a2_src, HEADS * HID),
                                 _expand_attn(a2_dst, HEADS * HID))
    acc2 = _EDGE_L12(src, dst, t2s, t2d, h2)
    x3 = _combine_kernel(acc2, P, Q, b2.reshape(1, -1), relu=True)

    h3, t3s, t3d = _dense_kernel(x3, W3, _expand_attn(a3_src, OUT),
                                 _expand_attn(a3_dst, OUT))
    acc3 = _EDGE_L3(src, dst, t3s, t3d, h3)
    P3, Q3 = _pq(OUT, 1, 80)
    out = _combine_kernel(acc3, P3, Q3, b3.reshape(1, -1), relu=False)

    return out[:N]


# speedup vs baseline: 30.7481x; 1.1785x over previous
import functools

import jax
import jax.numpy as jnp
from jax import lax
from jax.experimental import pallas as pl
from jax.experimental.pallas import tpu as pltpu
from jax.experimental.pallas import tpu_sc as plsc

N = 10000
IN = 128
HID = 32
HEADS = 4
OUT = 64
E = 320000

NC = 2
NS = 16
NW = NC * NS
LANES = 16

N_PAD = 10112
ROWS_PER_TILE = N_PAD // NS

B = 80
NE_REAL = E + N
CHUNK = -(-NE_REAL // (NW * 2 * B)) * 2 * B
E_PAD = CHUNK * NW
NBLOCKS = CHUNK // B


def _sc_edge_kernel(HC, H, R):
    mesh = plsc.VectorSubcoreMesh(core_axis_name="c", subcore_axis_name="s")
    C = HC // H
    nfull = ROWS_PER_TILE // B
    nrem = ROWS_PER_TILE % B

    @functools.partial(
        pl.kernel,
        mesh=mesh,
        compiler_params=pltpu.CompilerParams(use_tc_tiling_on_sc=False),
        out_type=jax.ShapeDtypeStruct((NC, N_PAD, R), jnp.float32),
        scratch_types=[
            pltpu.VMEM((2, B), jnp.int32),
            pltpu.VMEM((2, B), jnp.int32),
            pltpu.VMEM((2, B, 16), jnp.float32),
            pltpu.VMEM((2, B, 16), jnp.float32),
            pltpu.VMEM((2, B, HC), jnp.float32),
            pltpu.VMEM((B, R), jnp.float32),
            pltpu.VMEM_SHARED((N_PAD, R), jnp.float32),
            pltpu.SemaphoreType.DMA,
            pltpu.SemaphoreType.DMA,
            pltpu.SemaphoreType.DMA,
            pltpu.SemaphoreType.DMA,
            pltpu.SemaphoreType.DMA,
            pltpu.SemaphoreType.DMA,
        ],
    )
    def ker(src_hbm, dst_hbm, tsrc_hbm, tdst_hbm, h_hbm, out_hbm,
            srcb2, dstb2, tsb2, tdb2, hb2, msgb, acc, *sems):
        cid = lax.axis_index("c")
        sid = lax.axis_index("s")
        wid = sid * NC + cid

        zero16 = jnp.zeros((LANES,), jnp.float32)
        headmask = lax.iota(jnp.int32, LANES) < H

        @pl.loop(0, B)
        def _(e):
            @pl.loop(0, R, step=LANES)
            def _(cc):
                msgb[e, pl.ds(cc, LANES)] = zero16

        row0 = sid * ROWS_PER_TILE

        @pl.loop(0, nfull)
        def _(k):
            pltpu.sync_copy(msgb, acc.at[pl.ds(row0 + k * B, B)])

        if nrem:
            pltpu.sync_copy(msgb.at[pl.ds(0, nrem)],
                            acc.at[pl.ds(row0 + nfull * B, nrem)])

        plsc.subcore_barrier()

        base0 = wid * CHUNK

        def issue(p, blk):
            base = base0 + blk * B
            pltpu.sync_copy(src_hbm.at[pl.ds(base, B)], srcb2.at[p])
            pltpu.sync_copy(dst_hbm.at[pl.ds(base, B)], dstb2.at[p])
            pltpu.async_copy(tsrc_hbm.at[srcb2.at[p]], tsb2.at[p], sems[3 * p])
            pltpu.async_copy(tdst_hbm.at[dstb2.at[p]], tdb2.at[p],
                             sems[3 * p + 1])
            pltpu.async_copy(h_hbm.at[srcb2.at[p]], hb2.at[p], sems[3 * p + 2])

        def process(p):
            pltpu.make_async_copy(tsrc_hbm.at[srcb2.at[p]], tsb2.at[p],
                                  sems[3 * p]).wait()
            pltpu.make_async_copy(tdst_hbm.at[dstb2.at[p]], tdb2.at[p],
                                  sems[3 * p + 1]).wait()
            pltpu.make_async_copy(h_hbm.at[srcb2.at[p]], hb2.at[p],
                                  sems[3 * p + 2]).wait()

            @pl.loop(0, B)
            def _(e):
                ev = tsb2[p, e, pl.ds(0, LANES)] + tdb2[p, e, pl.ds(0, LANES)]
                ev = jnp.maximum(ev, 0.2 * ev)
                wv = jnp.exp(ev)
                msgb[e, pl.ds(HC, LANES)] = jnp.where(headmask, wv, 0.0)
                for h in range(H):
                    wh = wv[h]
                    for q in range(C // LANES):
                        coff = h * C + q * LANES
                        msgb[e, pl.ds(coff, LANES)] = (
                            hb2[p, e, pl.ds(coff, LANES)] * wh)

            pltpu.sync_copy(msgb, acc.at[dstb2.at[p]], add=True)

        issue(0, 0)
        issue(1, 1)

        @pl.loop(0, NBLOCKS // 2)
        def _(i2):
            for p in range(2):
                blk = i2 * 2 + p
                process(p)

                @pl.when(blk + 2 < NBLOCKS)
                def _():
                    issue(p, blk + 2)

        plsc.subcore_barrier()

        @pl.loop(0, nfull)
        def _(k):
            pltpu.sync_copy(acc.at[pl.ds(row0 + k * B, B)],
                            out_hbm.at[cid, pl.ds(row0 + k * B, B)])

        if nrem:
            pltpu.sync_copy(acc.at[pl.ds(row0 + nfull * B, nrem)],
                            out_hbm.at[cid, pl.ds(row0 + nfull * B, nrem)])

    return ker


def _dense_kernel(x_pad, W, Asrc16, Adst16):
    HCo = W.shape[1]

    def body(x_ref, w_ref, as_ref, ad_ref, h_ref, ts_ref, td_ref):
        h = jnp.dot(x_ref[...], w_ref[...],
                    preferred_element_type=jnp.float32,
                    precision=lax.Precision.HIGHEST)
        h_ref[...] = h
        ts_ref[...] = jnp.dot(h, as_ref[...],
                              preferred_element_type=jnp.float32,
                              precision=lax.Precision.HIGHEST)
        td_ref[...] = jnp.dot(h, ad_ref[...],
                              preferred_element_type=jnp.float32,
                              precision=lax.Precision.HIGHEST)

    Bn = N_PAD // 8
    IK = x_pad.shape[1]
    return pl.pallas_call(
        body,
        grid=(8,),
        in_specs=[
            pl.BlockSpec((Bn, IK), lambda i: (i, 0)),
            pl.BlockSpec((IK, HCo), lambda i: (0, 0)),
            pl.BlockSpec((HCo, 16), lambda i: (0, 0)),
            pl.BlockSpec((HCo, 16), lambda i: (0, 0)),
        ],
        out_specs=(
            pl.BlockSpec((Bn, HCo), lambda i: (i, 0)),
            pl.BlockSpec((Bn, 16), lambda i: (i, 0)),
            pl.BlockSpec((Bn, 16), lambda i: (i, 0)),
        ),
        out_shape=(
            jax.ShapeDtypeStruct((N_PAD, HCo), jnp.float32),
            jax.ShapeDtypeStruct((N_PAD, 16), jnp.float32),
            jax.ShapeDtypeStruct((N_PAD, 16), jnp.float32),
        ),
    )(x_pad, W, Asrc16, Adst16)


def _combine_kernel(acc, P, Q, b2d, relu):
    HCo = P.shape[1]

    def body(a_ref, p_ref, q_ref, b_ref, o_ref):
        a = a_ref[0] + a_ref[1]
        num = jnp.dot(a, p_ref[...], preferred_element_type=jnp.float32,
                      precision=lax.Precision.HIGHEST)
        den = jnp.dot(a, q_ref[...], preferred_element_type=jnp.float32,
                      precision=lax.Precision.HIGHEST) + 1e-16
        o = num / den + b_ref[...]
        if relu:
            o = jnp.maximum(o, 0.0)
        o_ref[...] = o

    R = P.shape[0]
    Bn = N_PAD // 8
    return pl.pallas_call(
        body,
        grid=(8,),
        in_specs=[
            pl.BlockSpec((2, Bn, R), lambda i: (0, i, 0)),
            pl.BlockSpec((R, HCo), lambda i: (0, 0)),
            pl.BlockSpec((R, HCo), lambda i: (0, 0)),
            pl.BlockSpec((1, HCo), lambda i: (0, 0)),
        ],
        out_specs=pl.BlockSpec((Bn, HCo), lambda i: (i, 0)),
        out_shape=jax.ShapeDtypeStruct((N_PAD, HCo), jnp.float32),
    )(acc, P, Q, b2d)


def _expand_attn(a, HC):
    H, C = a.shape
    mask = (jnp.arange(HC)[:, None] // C) == jnp.arange(H)[None, :]
    cols = jnp.where(mask, a.reshape(-1)[:, None], 0.0)
    return jnp.pad(cols, ((0, 0), (0, 16 - H)))


def _pq(HC, H, R):
    C = HC // H
    P = jnp.pad(jnp.eye(HC, dtype=jnp.float32), ((0, R - HC), (0, 0)))
    qrows = (jnp.arange(HC)[None, :] // C) == jnp.arange(H)[:, None]
    Q = jnp.zeros((R, HC), jnp.float32).at[HC:HC + H].set(qrows.astype(jnp.float32))
    return P, Q


_EDGE_L12 = _sc_edge_kernel(HEADS * HID, HEADS, 144)
_EDGE_L3 = _sc_edge_kernel(OUT, 1, 80)


def kernel(x, edge_index, W1, a1_src, a1_dst, b1, W2, a2_src, a2_dst, b2,
           W3, a3_src, a3_dst, b3):
    loop = jnp.arange(N, dtype=jnp.int32)
    src = jnp.concatenate([edge_index[0].astype(jnp.int32), loop])
    dst = jnp.concatenate([edge_index[1].astype(jnp.int32), loop])
    pad = jnp.full((E_PAD - NE_REAL,), N, jnp.int32)
    src = jnp.concatenate([src, pad])
    dst = jnp.concatenate([dst, pad])

    x_pad = jnp.pad(x, ((0, N_PAD - N), (0, 0)))

    h1, t1s, t1d = _dense_kernel(x_pad, W1, _expand_attn(a1_src, HEADS * HID),
                                 _expand_attn(a1_dst, HEADS * HID))
    acc1 = _EDGE_L12(src, dst, t1s, t1d, h1)
    P, Q = _pq(HEADS * HID, HEADS, 144)
    x2 = _combine_kernel(acc1, P, Q, b1.reshape(1, -1), relu=True)

    h2, t2s, t2d = _dense_kernel(x2, W2, _expand_attn(a2_src, HEADS * HID),
                                 _expand_attn(a2_dst, HEADS * HID))
    acc2 = _EDGE_L12(src, dst, t2s, t2d, h2)
    x3 = _combine_kernel(acc2, P, Q, b2.reshape(1, -1), relu=True)

    h3, t3s, t3d = _dense_kernel(x3, W3, _expand_attn(a3_src, OUT),
                                 _expand_attn(a3_dst, OUT))
    acc3 = _EDGE_L3(src, dst, t3s, t3d, h3)
    P3, Q3 = _pq(OUT, 1, 80)
    out = _combine_kernel(acc3, P3, Q3, b3.reshape(1, -1), relu=False)

    return out[:N]
